# Initial kernel scaffold; baseline (speedup 1.0000x reference)
import jax
import jax.numpy as jnp
from jax.experimental import pallas as pl


def kernel(node_features, edge_features, edge_index, num_nodes, params):
    raise NotImplementedError("write your pallas kernel here")


# speedup vs baseline: 2.1642x; 2.1642x over previous
import functools

import jax
import jax.numpy as jnp
from jax import lax
from jax.experimental import pallas as pl
from jax.experimental.pallas import tpu as pltpu
from jax.experimental.pallas import tpu_sc as plsc

_NC = 2
_NS = 16
_NW = _NC * _NS
_K = 80
_BE = 4000


def _relu(x):
    return jnp.maximum(x, 0.0)


def _wspec(shape):
    return pl.BlockSpec(shape, lambda i: tuple(0 for _ in shape))


def _dot(a, b):
    return jnp.dot(a, b, preferred_element_type=jnp.float32)


def _node_encoder_body(x_ref, w1_ref, b1_ref, w2_ref, b2_ref, o_ref):
    h = _relu(_dot(x_ref[...], w1_ref[...]) + b1_ref[...])
    o_ref[...] = _dot(h, w2_ref[...]) + b2_ref[...]


def _node_encoder(x, p):
    n = x.shape[0]
    do = p['W2'].shape[1]
    return pl.pallas_call(
        _node_encoder_body,
        out_shape=jax.ShapeDtypeStruct((n, do), jnp.float32),
    )(x, p['W1'], p['b1'].reshape(1, -1), p['W2'], p['b2'].reshape(1, -1))


def _node_update_body(ns_ref, a0_ref, a1_ref, w1n_ref, w1a_ref, b1_ref,
                      w2_ref, b2_ref, o_ref):
    agg = a0_ref[...] + a1_ref[...]
    h = _relu(_dot(ns_ref[...], w1n_ref[...]) + _dot(agg, w1a_ref[...])
              + b1_ref[...])
    o_ref[...] = _dot(h, w2_ref[...]) + b2_ref[...]


def _node_update(ns, agg_parts, p):
    n = ns.shape[0]
    d = ns.shape[1]
    w1n = p['W1'][:d]
    w1a = p['W1'][d:]
    return pl.pallas_call(
        _node_update_body,
        out_shape=jax.ShapeDtypeStruct((n, p['W2'].shape[1]), jnp.float32),
    )(ns, agg_parts[0], agg_parts[1], w1n, w1a, p['b1'].reshape(1, -1),
      p['W2'], p['b2'].reshape(1, -1))


def _messages_body(ef_ref, sg_ref, dg_ref, ew1_ref, eb1_ref, e2c_ref,
                   wa_ref, wb_ref, bc_ref, w2_ref, b2_ref, o_ref):
    h_e = _relu(_dot(ef_ref[...], ew1_ref[...]) + eb1_ref[...])
    pre = (_dot(sg_ref[...], wa_ref[...]) + _dot(dg_ref[...], wb_ref[...])
           + _dot(h_e, e2c_ref[...]) + bc_ref[...])
    o_ref[...] = _dot(_relu(pre), w2_ref[...]) + b2_ref[...]


def _messages(ef, gathered, pe, pm, d_node):
    e = ef.shape[0]
    de = ef.shape[1]
    nblk = e // _BE
    wa = pm['W1'][:d_node]
    wb = pm['W1'][d_node:2 * d_node]
    wc = pm['W1'][2 * d_node:]
    e2c = pe['W2'] @ wc
    bc = (pe['b2'] @ wc + pm['b1']).reshape(1, -1)
    dh = pm['W1'].shape[1]
    do = pm['W2'].shape[1]
    return pl.pallas_call(
        _messages_body,
        grid=(nblk,),
        in_specs=[
            pl.BlockSpec((_BE, de), lambda i: (i, 0)),
            pl.BlockSpec((_BE, d_node), lambda i: (i, 0)),
            pl.BlockSpec((_BE, d_node), lambda i: (i + nblk, 0)),
            _wspec(pe['W1'].shape),
            _wspec((1, pe['W1'].shape[1])),
            _wspec(e2c.shape),
            _wspec(wa.shape),
            _wspec(wb.shape),
            _wspec((1, dh)),
            _wspec(pm['W2'].shape),
            _wspec((1, do)),
        ],
        out_specs=pl.BlockSpec((_BE, do), lambda i: (i, 0)),
        out_shape=jax.ShapeDtypeStruct((e, do), jnp.float32),
    )(ef, gathered, gathered, pe['W1'], pe['b1'].reshape(1, -1), e2c,
      wa, wb, bc, pm['W2'], pm['b2'].reshape(1, -1))


def _edge_update_body(ef_ref, usg_ref, udg_ref, m_ref, ew1_ref, eb1_ref,
                      e2c_ref, wa_ref, wb_ref, wd_ref, bc_ref, w2_ref,
                      b2_ref, p1_ref, pb1_ref, p2_ref, pb2_ref,
                      oue_ref, ol_ref):
    h_e = _relu(_dot(ef_ref[...], ew1_ref[...]) + eb1_ref[...])
    pre = (_dot(usg_ref[...], wa_ref[...]) + _dot(udg_ref[...], wb_ref[...])
           + _dot(h_e, e2c_ref[...]) + _dot(m_ref[...], wd_ref[...])
           + bc_ref[...])
    ue = _dot(_relu(pre), w2_ref[...]) + b2_ref[...]
    oue_ref[...] = ue
    hp = _relu(_dot(ue, p1_ref[...]) + pb1_ref[...])
    ol_ref[...] = _dot(hp, p2_ref[...]) + pb2_ref[...]


def _edge_update(ef, gathered, msgs, pe, pu, pp, d_node):
    e = ef.shape[0]
    de = ef.shape[1]
    nblk = e // _BE
    dm = msgs.shape[1]
    wa = pu['W1'][:d_node]
    wb = pu['W1'][d_node:2 * d_node]
    wc = pu['W1'][2 * d_node:2 * d_node + pe['W2'].shape[1]]
    wd = pu['W1'][2 * d_node + pe['W2'].shape[1]:]
    e2c = pe['W2'] @ wc
    bc = (pe['b2'] @ wc + pu['b1']).reshape(1, -1)
    dh = pu['W1'].shape[1]
    do = pu['W2'].shape[1]
    return pl.pallas_call(
        _edge_update_body,
        grid=(nblk,),
        in_specs=[
            pl.BlockSpec((_BE, de), lambda i: (i, 0)),
            pl.BlockSpec((_BE, d_node), lambda i: (i, 0)),
            pl.BlockSpec((_BE, d_node), lambda i: (i + nblk, 0)),
            pl.BlockSpec((_BE, dm), lambda i: (i, 0)),
            _wspec(pe['W1'].shape),
            _wspec((1, pe['W1'].shape[1])),
            _wspec(e2c.shape),
            _wspec(wa.shape),
            _wspec(wb.shape),
            _wspec(wd.shape),
            _wspec((1, dh)),
            _wspec(pu['W2'].shape),
            _wspec((1, do)),
            _wspec(pp['W1'].shape),
            _wspec((1, pp['W1'].shape[1])),
            _wspec(pp['W2'].shape),
            _wspec((1, 1)),
        ],
        out_specs=[
            pl.BlockSpec((_BE, do), lambda i: (i, 0)),
            pl.BlockSpec((_BE, 1), lambda i: (i, 0)),
        ],
        out_shape=[
            jax.ShapeDtypeStruct((e, do), jnp.float32),
            jax.ShapeDtypeStruct((e, 1), jnp.float32),
        ],
    )(ef, gathered, gathered, msgs, pe['W1'], pe['b1'].reshape(1, -1), e2c,
      wa, wb, wd, bc, pu['W2'], pu['b2'].reshape(1, -1),
      pp['W1'], pp['b1'].reshape(1, -1), pp['W2'], pp['b2'].reshape(1, 1))


def _sc_gather(table, idx):
    b = idx.shape[0]
    d = table.shape[1]
    b_per_w = b // _NW
    chunks = b_per_w // _K
    mesh = plsc.VectorSubcoreMesh(core_axis_name="c", subcore_axis_name="s")

    @functools.partial(
        pl.kernel,
        mesh=mesh,
        out_type=jax.ShapeDtypeStruct((b, d), jnp.float32),
        scratch_types=[
            pltpu.VMEM((_K,), jnp.int32),
            pltpu.VMEM((_K, d), jnp.float32),
        ],
    )
    def k(table_hbm, idx_hbm, out_hbm, idx_v, rows_v):
        wid = lax.axis_index("s") * _NC + lax.axis_index("c")
        base = wid * b_per_w

        def body(j, carry):
            off = base + j * _K
            pltpu.sync_copy(idx_hbm.at[pl.ds(off, _K)], idx_v)
            pltpu.sync_copy(table_hbm.at[idx_v], rows_v)
            pltpu.sync_copy(rows_v, out_hbm.at[pl.ds(off, _K)])
            return carry

        lax.fori_loop(0, chunks, body, 0)

    return k(table, idx)


def _sc_scatter_add(msgs, idx, zeros_init):
    e, d = msgs.shape
    n = zeros_init.shape[0]
    per_core = e // _NC
    per_tile = per_core // _NS
    chunks = per_tile // _K
    nz = max(t for t in range(1, _NS + 1) if n % t == 0 and (n // t) % 8 == 0)
    stripe_rows = n // nz
    mesh = plsc.VectorSubcoreMesh(core_axis_name="c", subcore_axis_name="s")

    @functools.partial(
        pl.kernel,
        mesh=mesh,
        out_type=jax.ShapeDtypeStruct((_NC, n, d), jnp.float32),
        scratch_types=[
            pltpu.VMEM((_K,), jnp.int32),
            pltpu.VMEM((_K, d), jnp.float32),
            pltpu.VMEM_SHARED((n, d), jnp.float32),
        ],
    )
    def k(msg_hbm, idx_hbm, zero_hbm, out_hbm, idx_v, rows_v, acc):
        cid = lax.axis_index("c")
        sid = lax.axis_index("s")
        stripe = pl.ds(sid * stripe_rows, stripe_rows)

        @pl.when(sid < nz)
        def _init():
            pltpu.sync_copy(zero_hbm.at[stripe], acc.at[stripe])

        plsc.subcore_barrier()
        base = cid * per_core + sid * per_tile

        def body(j, carry):
            off = base + j * _K
            pltpu.sync_copy(idx_hbm.at[pl.ds(off, _K)], idx_v)
            pltpu.sync_copy(msg_hbm.at[pl.ds(off, _K)], rows_v)
            pltpu.sync_copy(rows_v, acc.at[idx_v], add=True)
            return carry

        lax.fori_loop(0, chunks, body, 0)
        plsc.subcore_barrier()

        @pl.when(sid < nz)
        def _writeback():
            pltpu.sync_copy(acc.at[stripe], out_hbm.at[cid].at[stripe])

    return k(msgs, idx, zeros_init)


def kernel(node_features, edge_features, edge_index, num_nodes, params):
    n, d_node = node_features.shape
    e = edge_features.shape[0]

    idx_all = edge_index.reshape(-1).astype(jnp.int32)
    scat_idx = (edge_index[1] % num_nodes).astype(jnp.int32)

    node_state = _node_encoder(node_features, params['node_encoder'])

    gathered = _sc_gather(node_state, idx_all)
    messages = _messages(edge_features, gathered, params['edge_encoder'],
                         params['message_encoder'], d_node)

    zeros_init = jnp.zeros((n, messages.shape[1]), jnp.float32)
    agg_parts = _sc_scatter_add(messages, scat_idx, zeros_init)

    updated_node_state = _node_update(node_state, agg_parts,
                                      params['node_updater'])

    gathered2 = _sc_gather(updated_node_state, idx_all)
    updated_edge_state, logits = _edge_update(
        edge_features, gathered2, messages, params['edge_encoder'],
        params['edge_updater'], params['predictor'], d_node)

    return (logits.reshape(e), updated_node_state, updated_edge_state,
            messages)


# speedup vs baseline: 3.3429x; 1.5447x over previous
import functools

import jax
import jax.numpy as jnp
from jax import lax
from jax.experimental import pallas as pl
from jax.experimental.pallas import tpu as pltpu
from jax.experimental.pallas import tpu_sc as plsc

_NC = 2
_NS = 16
_NW = _NC * _NS
_K = 80
_BE = 4000


def _relu(x):
    return jnp.maximum(x, 0.0)


def _wspec(shape):
    return pl.BlockSpec(shape, lambda i: tuple(0 for _ in shape))


def _dot(a, b):
    return jnp.dot(a, b, preferred_element_type=jnp.float32)


def _node_encoder_body(x_ref, w1_ref, b1_ref, w2_ref, b2_ref, o_ref):
    h = _relu(_dot(x_ref[...], w1_ref[...]) + b1_ref[...])
    o_ref[...] = _dot(h, w2_ref[...]) + b2_ref[...]


def _node_encoder(x, p):
    n = x.shape[0]
    do = p['W2'].shape[1]
    return pl.pallas_call(
        _node_encoder_body,
        out_shape=jax.ShapeDtypeStruct((n, do), jnp.float32),
    )(x, p['W1'], p['b1'].reshape(1, -1), p['W2'], p['b2'].reshape(1, -1))


def _node_update_body(ns_ref, a0_ref, a1_ref, w1n_ref, w1a_ref, b1_ref,
                      w2_ref, b2_ref, o_ref):
    agg = a0_ref[...] + a1_ref[...]
    h = _relu(_dot(ns_ref[...], w1n_ref[...]) + _dot(agg, w1a_ref[...])
              + b1_ref[...])
    o_ref[...] = _dot(h, w2_ref[...]) + b2_ref[...]


def _node_update(ns, agg_parts, p):
    n = ns.shape[0]
    d = ns.shape[1]
    w1n = p['W1'][:d]
    w1a = p['W1'][d:]
    return pl.pallas_call(
        _node_update_body,
        out_shape=jax.ShapeDtypeStruct((n, p['W2'].shape[1]), jnp.float32),
    )(ns, agg_parts[0], agg_parts[1], w1n, w1a, p['b1'].reshape(1, -1),
      p['W2'], p['b2'].reshape(1, -1))


def _messages_body(ef_ref, sg_ref, dg_ref, ew1_ref, eb1_ref, e2c_ref,
                   wa_ref, wb_ref, bc_ref, w2_ref, b2_ref, o_ref):
    h_e = _relu(_dot(ef_ref[...], ew1_ref[...]) + eb1_ref[...])
    pre = (_dot(sg_ref[...], wa_ref[...]) + _dot(dg_ref[...], wb_ref[...])
           + _dot(h_e, e2c_ref[...]) + bc_ref[...])
    o_ref[...] = _dot(_relu(pre), w2_ref[...]) + b2_ref[...]


def _messages(ef, gathered, pe, pm, d_node):
    e = ef.shape[0]
    de = ef.shape[1]
    nblk = e // _BE
    wa = pm['W1'][:d_node]
    wb = pm['W1'][d_node:2 * d_node]
    wc = pm['W1'][2 * d_node:]
    e2c = pe['W2'] @ wc
    bc = (pe['b2'] @ wc + pm['b1']).reshape(1, -1)
    dh = pm['W1'].shape[1]
    do = pm['W2'].shape[1]
    return pl.pallas_call(
        _messages_body,
        grid=(nblk,),
        in_specs=[
            pl.BlockSpec((_BE, de), lambda i: (i, 0)),
            pl.BlockSpec((_BE, d_node), lambda i: (i, 0)),
            pl.BlockSpec((_BE, d_node), lambda i: (i + nblk, 0)),
            _wspec(pe['W1'].shape),
            _wspec((1, pe['W1'].shape[1])),
            _wspec(e2c.shape),
            _wspec(wa.shape),
            _wspec(wb.shape),
            _wspec((1, dh)),
            _wspec(pm['W2'].shape),
            _wspec((1, do)),
        ],
        out_specs=pl.BlockSpec((_BE, do), lambda i: (i, 0)),
        out_shape=jax.ShapeDtypeStruct((e, do), jnp.float32),
    )(ef, gathered, gathered, pe['W1'], pe['b1'].reshape(1, -1), e2c,
      wa, wb, bc, pm['W2'], pm['b2'].reshape(1, -1))


def _edge_update_body(ef_ref, usg_ref, udg_ref, m_ref, ew1_ref, eb1_ref,
                      e2c_ref, wa_ref, wb_ref, wd_ref, bc_ref, w2_ref,
                      b2_ref, p1_ref, pb1_ref, p2_ref, pb2_ref,
                      oue_ref, ol_ref):
    h_e = _relu(_dot(ef_ref[...], ew1_ref[...]) + eb1_ref[...])
    pre = (_dot(usg_ref[...], wa_ref[...]) + _dot(udg_ref[...], wb_ref[...])
           + _dot(h_e, e2c_ref[...]) + _dot(m_ref[...], wd_ref[...])
           + bc_ref[...])
    ue = _dot(_relu(pre), w2_ref[...]) + b2_ref[...]
    oue_ref[...] = ue
    hp = _relu(_dot(ue, p1_ref[...]) + pb1_ref[...])
    ol_ref[...] = _dot(hp, p2_ref[...]) + pb2_ref[...]


def _edge_update(ef, gathered, msgs, pe, pu, pp, d_node):
    e = ef.shape[0]
    de = ef.shape[1]
    nblk = e // _BE
    dm = msgs.shape[1]
    wa = pu['W1'][:d_node]
    wb = pu['W1'][d_node:2 * d_node]
    wc = pu['W1'][2 * d_node:2 * d_node + pe['W2'].shape[1]]
    wd = pu['W1'][2 * d_node + pe['W2'].shape[1]:]
    e2c = pe['W2'] @ wc
    bc = (pe['b2'] @ wc + pu['b1']).reshape(1, -1)
    dh = pu['W1'].shape[1]
    do = pu['W2'].shape[1]
    return pl.pallas_call(
        _edge_update_body,
        grid=(nblk,),
        in_specs=[
            pl.BlockSpec((_BE, de), lambda i: (i, 0)),
            pl.BlockSpec((_BE, d_node), lambda i: (i, 0)),
            pl.BlockSpec((_BE, d_node), lambda i: (i + nblk, 0)),
            pl.BlockSpec((_BE, dm), lambda i: (i, 0)),
            _wspec(pe['W1'].shape),
            _wspec((1, pe['W1'].shape[1])),
            _wspec(e2c.shape),
            _wspec(wa.shape),
            _wspec(wb.shape),
            _wspec(wd.shape),
            _wspec((1, dh)),
            _wspec(pu['W2'].shape),
            _wspec((1, do)),
            _wspec(pp['W1'].shape),
            _wspec((1, pp['W1'].shape[1])),
            _wspec(pp['W2'].shape),
            _wspec((1, 1)),
        ],
        out_specs=[
            pl.BlockSpec((_BE, do), lambda i: (i, 0)),
            pl.BlockSpec((_BE, 1), lambda i: (i, 0)),
        ],
        out_shape=[
            jax.ShapeDtypeStruct((e, do), jnp.float32),
            jax.ShapeDtypeStruct((e, 1), jnp.float32),
        ],
    )(ef, gathered, gathered, msgs, pe['W1'], pe['b1'].reshape(1, -1), e2c,
      wa, wb, wd, bc, pu['W2'], pu['b2'].reshape(1, -1),
      pp['W1'], pp['b1'].reshape(1, -1), pp['W2'], pp['b2'].reshape(1, 1))


_RB = 5


def _sc_gather(table, idx):
    b = idx.shape[0]
    d = table.shape[1]
    b_per_w = b // _NW
    chunks = b_per_w // _K
    groups = chunks // _RB
    mesh = plsc.VectorSubcoreMesh(core_axis_name="c", subcore_axis_name="s")

    @functools.partial(
        pl.kernel,
        mesh=mesh,
        out_type=jax.ShapeDtypeStruct((b, d), jnp.float32),
        scratch_types=(
            [pltpu.VMEM((b_per_w,), jnp.int32)]
            + [pltpu.VMEM((_K, d), jnp.float32)] * _RB
            + [pltpu.SemaphoreType.DMA] * (2 * _RB)
        ),
    )
    def k(table_hbm, idx_hbm, out_hbm, idx_v, *bufs):
        rows = bufs[:_RB]
        sg = bufs[_RB:2 * _RB]
        sw = bufs[2 * _RB:]
        wid = lax.axis_index("s") * _NC + lax.axis_index("c")
        base = wid * b_per_w
        pltpu.sync_copy(idx_hbm.at[pl.ds(base, b_per_w)], idx_v)

        def body(g, carry):
            descs = []
            for bb in range(_RB):
                c = g * _RB + bb
                off = base + c * _K

                @pl.when(g > 0)
                def _drain_wb(bb=bb, off=off):
                    pltpu.make_async_copy(
                        rows[bb], out_hbm.at[pl.ds(off, _K)], sw[bb]).wait()

                descs.append(pltpu.async_copy(
                    table_hbm.at[idx_v.at[pl.ds(c * _K, _K)]], rows[bb],
                    sg[bb]))
            for bb in range(_RB):
                c = g * _RB + bb
                off = base + c * _K
                descs[bb].wait()
                pltpu.async_copy(rows[bb], out_hbm.at[pl.ds(off, _K)],
                                 sw[bb])
            return carry

        lax.fori_loop(0, groups, body, 0)
        for bb in range(_RB):
            pltpu.make_async_copy(
                rows[bb], out_hbm.at[pl.ds(base, _K)], sw[bb]).wait()

    return k(table, idx)


def _sc_scatter_add(msgs, idx, zeros_init):
    e, d = msgs.shape
    n = zeros_init.shape[0]
    per_core = e // _NC
    per_tile = per_core // _NS
    k_sc = 40
    chunks = per_tile // k_sc
    nz = max(t for t in range(1, _NS + 1) if n % t == 0 and (n // t) % 8 == 0)
    stripe_rows = n // nz
    mesh = plsc.VectorSubcoreMesh(core_axis_name="c", subcore_axis_name="s")

    groups = chunks // _RB

    @functools.partial(
        pl.kernel,
        mesh=mesh,
        out_type=jax.ShapeDtypeStruct((_NC, n, d), jnp.float32),
        scratch_types=(
            [pltpu.VMEM((k_sc,), jnp.int32)] * _RB
            + [pltpu.VMEM((k_sc, d), jnp.float32)] * _RB
            + [pltpu.VMEM_SHARED((n, d), jnp.float32)]
            + [pltpu.SemaphoreType.DMA] * (3 * _RB)
        ),
    )
    def k(msg_hbm, idx_hbm, zero_hbm, out_hbm, *bufs):
        idxs = bufs[:_RB]
        rows = bufs[_RB:2 * _RB]
        acc = bufs[2 * _RB]
        si = bufs[2 * _RB + 1:2 * _RB + 1 + _RB]
        sm = bufs[2 * _RB + 1 + _RB:2 * _RB + 1 + 2 * _RB]
        sa = bufs[2 * _RB + 1 + 2 * _RB:]
        cid = lax.axis_index("c")
        sid = lax.axis_index("s")
        stripe = pl.ds(sid * stripe_rows, stripe_rows)

        @pl.when(sid < nz)
        def _init():
            pltpu.sync_copy(zero_hbm.at[stripe], acc.at[stripe])

        plsc.subcore_barrier()
        base = cid * per_core + sid * per_tile

        def body(g, carry):
            di = []
            dm = []
            for bb in range(_RB):
                c = g * _RB + bb
                off = base + c * k_sc

                @pl.when(g > 0)
                def _drain_add(bb=bb):
                    pltpu.make_async_copy(
                        rows[bb], acc.at[idxs[bb]], sa[bb]).wait()

                di.append(pltpu.async_copy(
                    idx_hbm.at[pl.ds(off, k_sc)], idxs[bb], si[bb]))
                dm.append(pltpu.async_copy(
                    msg_hbm.at[pl.ds(off, k_sc)], rows[bb], sm[bb]))
            for bb in range(_RB):
                di[bb].wait()
                dm[bb].wait()
                pltpu.async_copy(rows[bb], acc.at[idxs[bb]], sa[bb],
                                 add=True)
            return carry

        lax.fori_loop(0, groups, body, 0)
        for bb in range(_RB):
            pltpu.make_async_copy(rows[bb], acc.at[idxs[bb]], sa[bb]).wait()
        plsc.subcore_barrier()

        @pl.when(sid < nz)
        def _writeback():
            pltpu.sync_copy(acc.at[stripe], out_hbm.at[cid].at[stripe])

    return k(msgs, idx, zeros_init)


def kernel(node_features, edge_features, edge_index, num_nodes, params):
    n, d_node = node_features.shape
    e = edge_features.shape[0]

    idx_all = edge_index.reshape(-1).astype(jnp.int32)
    scat_idx = (edge_index[1] % num_nodes).astype(jnp.int32)

    node_state = _node_encoder(node_features, params['node_encoder'])

    gathered = _sc_gather(node_state, idx_all)
    messages = _messages(edge_features, gathered, params['edge_encoder'],
                         params['message_encoder'], d_node)

    zeros_init = jnp.zeros((n, messages.shape[1]), jnp.float32)
    agg_parts = _sc_scatter_add(messages, scat_idx, zeros_init)

    updated_node_state = _node_update(node_state, agg_parts,
                                      params['node_updater'])

    gathered2 = _sc_gather(updated_node_state, idx_all)
    updated_edge_state, logits = _edge_update(
        edge_features, gathered2, messages, params['edge_encoder'],
        params['edge_updater'], params['predictor'], d_node)

    return (logits.reshape(e), updated_node_state, updated_edge_state,
            messages)


# speedup vs baseline: 3.3745x; 1.0094x over previous
import functools

import jax
import jax.numpy as jnp
from jax import lax
from jax.experimental import pallas as pl
from jax.experimental.pallas import tpu as pltpu
from jax.experimental.pallas import tpu_sc as plsc

_NC = 2
_NS = 16
_NW = _NC * _NS
_K = 80
_BE = 4000


def _relu(x):
    return jnp.maximum(x, 0.0)


def _wspec(shape):
    return pl.BlockSpec(shape, lambda i: tuple(0 for _ in shape))


def _dot(a, b):
    return jnp.dot(a, b, preferred_element_type=jnp.float32)


def _node_encoder_body(x_ref, w1_ref, b1_ref, w2_ref, b2_ref, o_ref):
    h = _relu(_dot(x_ref[...], w1_ref[...]) + b1_ref[...])
    o_ref[...] = _dot(h, w2_ref[...]) + b2_ref[...]


def _node_encoder(x, p):
    n = x.shape[0]
    do = p['W2'].shape[1]
    return pl.pallas_call(
        _node_encoder_body,
        out_shape=jax.ShapeDtypeStruct((n, do), jnp.float32),
    )(x, p['W1'], p['b1'].reshape(1, -1), p['W2'], p['b2'].reshape(1, -1))


def _node_update_body(ns_ref, a0_ref, a1_ref, w1n_ref, w1a_ref, b1_ref,
                      w2_ref, b2_ref, o_ref):
    agg = a0_ref[...] + a1_ref[...]
    h = _relu(_dot(ns_ref[...], w1n_ref[...]) + _dot(agg, w1a_ref[...])
              + b1_ref[...])
    o_ref[...] = _dot(h, w2_ref[...]) + b2_ref[...]


def _node_update(ns, agg_parts, p):
    n = ns.shape[0]
    d = ns.shape[1]
    w1n = p['W1'][:d]
    w1a = p['W1'][d:]
    do = p['W2'].shape[1]
    return pl.pallas_call(
        _node_update_body,
        out_shape=jax.ShapeDtypeStruct((n, do), jnp.float32),
    )(ns, agg_parts[0], agg_parts[1], w1n, w1a, p['b1'].reshape(1, -1),
      p['W2'], p['b2'].reshape(1, -1))


def _messages_body(ef_ref, sg_ref, dg_ref, ew1_ref, eb1_ref, e2c_ref,
                   wa_ref, wb_ref, bc_ref, w2_ref, b2_ref, o_ref, obf_ref):
    h_e = _relu(_dot(ef_ref[...], ew1_ref[...]) + eb1_ref[...])
    sg = sg_ref[...].astype(jnp.bfloat16)
    dg = dg_ref[...].astype(jnp.bfloat16)
    pre = (_dot(sg, wa_ref[...]) + _dot(dg, wb_ref[...])
           + _dot(h_e, e2c_ref[...]) + bc_ref[...])
    h2 = _relu(pre).astype(jnp.bfloat16)
    m = _dot(h2, w2_ref[...]) + b2_ref[...]
    o_ref[...] = m
    obf_ref[...] = m.astype(jnp.bfloat16)


def _messages(ef, gathered, pe, pm, d_node):
    e = ef.shape[0]
    de = ef.shape[1]
    nblk = e // _BE
    wa = pm['W1'][:d_node].astype(jnp.bfloat16)
    wb = pm['W1'][d_node:2 * d_node].astype(jnp.bfloat16)
    wc = pm['W1'][2 * d_node:]
    e2c = pe['W2'] @ wc
    bc = (pe['b2'] @ wc + pm['b1']).reshape(1, -1)
    w2 = pm['W2'].astype(jnp.bfloat16)
    dh = pm['W1'].shape[1]
    do = pm['W2'].shape[1]
    return pl.pallas_call(
        _messages_body,
        grid=(nblk,),
        in_specs=[
            pl.BlockSpec((_BE, de), lambda i: (i, 0)),
            pl.BlockSpec((_BE, d_node), lambda i: (i, 0)),
            pl.BlockSpec((_BE, d_node), lambda i: (i + nblk, 0)),
            _wspec(pe['W1'].shape),
            _wspec((1, pe['W1'].shape[1])),
            _wspec(e2c.shape),
            _wspec(wa.shape),
            _wspec(wb.shape),
            _wspec((1, dh)),
            _wspec(w2.shape),
            _wspec((1, do)),
        ],
        out_specs=[pl.BlockSpec((_BE, do), lambda i: (i, 0)),
                   pl.BlockSpec((_BE, do), lambda i: (i, 0))],
        out_shape=[jax.ShapeDtypeStruct((e, do), jnp.float32),
                   jax.ShapeDtypeStruct((e, do), jnp.bfloat16)],
    )(ef, gathered, gathered, pe['W1'], pe['b1'].reshape(1, -1), e2c,
      wa, wb, bc, w2, pm['b2'].reshape(1, -1))


def _edge_update_body(ef_ref, usg_ref, udg_ref, m_ref, ew1_ref, eb1_ref,
                      e2c_ref, wa_ref, wb_ref, wd_ref, bc_ref, w2_ref,
                      b2_ref, p1_ref, pb1_ref, p2_ref, pb2_ref,
                      oue_ref, ol_ref):
    h_e = _relu(_dot(ef_ref[...], ew1_ref[...]) + eb1_ref[...])
    usg = usg_ref[...].astype(jnp.bfloat16)
    udg = udg_ref[...].astype(jnp.bfloat16)
    pre = (_dot(usg, wa_ref[...]) + _dot(udg, wb_ref[...])
           + _dot(h_e, e2c_ref[...]) + _dot(m_ref[...], wd_ref[...])
           + bc_ref[...])
    h2 = _relu(pre).astype(jnp.bfloat16)
    ue = _dot(h2, w2_ref[...]) + b2_ref[...]
    oue_ref[...] = ue
    hp = _relu(_dot(ue.astype(jnp.bfloat16), p1_ref[...]) + pb1_ref[...])
    ol_ref[...] = _dot(hp.astype(jnp.bfloat16), p2_ref[...]) + pb2_ref[...]


def _edge_update(ef, gathered, msgs, pe, pu, pp, d_node):
    e = ef.shape[0]
    de = ef.shape[1]
    nblk = e // _BE
    dm = msgs.shape[1]
    wa = pu['W1'][:d_node].astype(jnp.bfloat16)
    wb = pu['W1'][d_node:2 * d_node].astype(jnp.bfloat16)
    wc = pu['W1'][2 * d_node:2 * d_node + pe['W2'].shape[1]]
    wd = pu['W1'][2 * d_node + pe['W2'].shape[1]:].astype(jnp.bfloat16)
    w2u = pu['W2'].astype(jnp.bfloat16)
    p1 = pp['W1'].astype(jnp.bfloat16)
    p2 = pp['W2'].astype(jnp.bfloat16)
    e2c = pe['W2'] @ wc
    bc = (pe['b2'] @ wc + pu['b1']).reshape(1, -1)
    dh = pu['W1'].shape[1]
    do = pu['W2'].shape[1]
    return pl.pallas_call(
        _edge_update_body,
        grid=(nblk,),
        in_specs=[
            pl.BlockSpec((_BE, de), lambda i: (i, 0)),
            pl.BlockSpec((_BE, d_node), lambda i: (i, 0)),
            pl.BlockSpec((_BE, d_node), lambda i: (i + nblk, 0)),
            pl.BlockSpec((_BE, dm), lambda i: (i, 0)),
            _wspec(pe['W1'].shape),
            _wspec((1, pe['W1'].shape[1])),
            _wspec(e2c.shape),
            _wspec(wa.shape),
            _wspec(wb.shape),
            _wspec(wd.shape),
            _wspec((1, dh)),
            _wspec(w2u.shape),
            _wspec((1, do)),
            _wspec(p1.shape),
            _wspec((1, p1.shape[1])),
            _wspec(p2.shape),
            _wspec((1, 1)),
        ],
        out_specs=[
            pl.BlockSpec((_BE, do), lambda i: (i, 0)),
            pl.BlockSpec((_BE, 1), lambda i: (i, 0)),
        ],
        out_shape=[
            jax.ShapeDtypeStruct((e, do), jnp.float32),
            jax.ShapeDtypeStruct((e, 1), jnp.float32),
        ],
    )(ef, gathered, gathered, msgs, pe['W1'], pe['b1'].reshape(1, -1), e2c,
      wa, wb, wd, bc, w2u, pu['b2'].reshape(1, -1),
      p1, pp['b1'].reshape(1, -1), p2, pp['b2'].reshape(1, 1))


_RB = 5


def _sc_gather(table, idx):
    b = idx.shape[0]
    d = table.shape[1]
    dt = table.dtype
    b_per_w = b // _NW
    chunks = b_per_w // _K
    groups = chunks // _RB
    mesh = plsc.VectorSubcoreMesh(core_axis_name="c", subcore_axis_name="s")

    @functools.partial(
        pl.kernel,
        mesh=mesh,
        out_type=jax.ShapeDtypeStruct((b, d), dt),
        scratch_types=(
            [pltpu.VMEM((b_per_w,), jnp.int32)]
            + [pltpu.VMEM((_K, d), dt)] * _RB
            + [pltpu.SemaphoreType.DMA] * (2 * _RB)
        ),
    )
    def k(table_hbm, idx_hbm, out_hbm, idx_v, *bufs):
        rows = bufs[:_RB]
        sg = bufs[_RB:2 * _RB]
        sw = bufs[2 * _RB:]
        wid = lax.axis_index("s") * _NC + lax.axis_index("c")
        base = wid * b_per_w
        pltpu.sync_copy(idx_hbm.at[pl.ds(base, b_per_w)], idx_v)

        def body(g, carry):
            descs = []
            for bb in range(_RB):
                c = g * _RB + bb
                off = base + c * _K

                @pl.when(g > 0)
                def _drain_wb(bb=bb, off=off):
                    pltpu.make_async_copy(
                        rows[bb], out_hbm.at[pl.ds(off, _K)], sw[bb]).wait()

                descs.append(pltpu.async_copy(
                    table_hbm.at[idx_v.at[pl.ds(c * _K, _K)]], rows[bb],
                    sg[bb]))
            for bb in range(_RB):
                c = g * _RB + bb
                off = base + c * _K
                descs[bb].wait()
                pltpu.async_copy(rows[bb], out_hbm.at[pl.ds(off, _K)],
                                 sw[bb])
            return carry

        lax.fori_loop(0, groups, body, 0)
        for bb in range(_RB):
            pltpu.make_async_copy(
                rows[bb], out_hbm.at[pl.ds(base, _K)], sw[bb]).wait()

    return k(table, idx)


def _sc_scatter_add(msgs, idx, zeros_init):
    e, d = msgs.shape
    n = zeros_init.shape[0]
    per_core = e // _NC
    per_tile = per_core // _NS
    k_sc = 40
    chunks = per_tile // k_sc
    nz = max(t for t in range(1, _NS + 1) if n % t == 0 and (n // t) % 8 == 0)
    stripe_rows = n // nz
    mesh = plsc.VectorSubcoreMesh(core_axis_name="c", subcore_axis_name="s")

    groups = chunks // _RB

    @functools.partial(
        pl.kernel,
        mesh=mesh,
        out_type=jax.ShapeDtypeStruct((_NC, n, d), jnp.float32),
        scratch_types=(
            [pltpu.VMEM((k_sc,), jnp.int32)] * _RB
            + [pltpu.VMEM((k_sc, d), jnp.float32)] * _RB
            + [pltpu.VMEM_SHARED((n, d), jnp.float32)]
            + [pltpu.SemaphoreType.DMA] * (3 * _RB)
        ),
    )
    def k(msg_hbm, idx_hbm, zero_hbm, out_hbm, *bufs):
        idxs = bufs[:_RB]
        rows = bufs[_RB:2 * _RB]
        acc = bufs[2 * _RB]
        si = bufs[2 * _RB + 1:2 * _RB + 1 + _RB]
        sm = bufs[2 * _RB + 1 + _RB:2 * _RB + 1 + 2 * _RB]
        sa = bufs[2 * _RB + 1 + 2 * _RB:]
        cid = lax.axis_index("c")
        sid = lax.axis_index("s")
        stripe = pl.ds(sid * stripe_rows, stripe_rows)

        @pl.when(sid < nz)
        def _init():
            pltpu.sync_copy(zero_hbm.at[stripe], acc.at[stripe])

        plsc.subcore_barrier()
        base = cid * per_core + sid * per_tile

        def body(g, carry):
            di = []
            dm = []
            for bb in range(_RB):
                c = g * _RB + bb
                off = base + c * k_sc

                @pl.when(g > 0)
                def _drain_add(bb=bb):
                    pltpu.make_async_copy(
                        rows[bb], acc.at[idxs[bb]], sa[bb]).wait()

                di.append(pltpu.async_copy(
                    idx_hbm.at[pl.ds(off, k_sc)], idxs[bb], si[bb]))
                dm.append(pltpu.async_copy(
                    msg_hbm.at[pl.ds(off, k_sc)], rows[bb], sm[bb]))
            for bb in range(_RB):
                di[bb].wait()
                dm[bb].wait()
                pltpu.async_copy(rows[bb], acc.at[idxs[bb]], sa[bb],
                                 add=True)
            return carry

        lax.fori_loop(0, groups, body, 0)
        for bb in range(_RB):
            pltpu.make_async_copy(rows[bb], acc.at[idxs[bb]], sa[bb]).wait()
        plsc.subcore_barrier()

        @pl.when(sid < nz)
        def _writeback():
            pltpu.sync_copy(acc.at[stripe], out_hbm.at[cid].at[stripe])

    return k(msgs, idx, zeros_init)


def kernel(node_features, edge_features, edge_index, num_nodes, params):
    n, d_node = node_features.shape
    e = edge_features.shape[0]

    idx_all = edge_index.reshape(-1).astype(jnp.int32)
    scat_idx = (edge_index[1] % num_nodes).astype(jnp.int32)

    node_state = _node_encoder(node_features, params['node_encoder'])

    gathered = _sc_gather(node_state, idx_all)
    messages, messages_bf = _messages(edge_features, gathered,
                                      params['edge_encoder'],
                                      params['message_encoder'], d_node)

    zeros_init = jnp.zeros((n, messages.shape[1]), jnp.float32)
    agg_parts = _sc_scatter_add(messages, scat_idx, zeros_init)

    updated_node_state = _node_update(node_state, agg_parts,
                                      params['node_updater'])

    gathered2 = _sc_gather(updated_node_state, idx_all)
    updated_edge_state, logits = _edge_update(
        edge_features, gathered2, messages_bf, params['edge_encoder'],
        params['edge_updater'], params['predictor'], d_node)

    return (logits.reshape(e), updated_node_state, updated_edge_state,
            messages)


# speedup vs baseline: 3.3817x; 1.0021x over previous
import functools

import jax
import jax.numpy as jnp
from jax import lax
from jax.experimental import pallas as pl
from jax.experimental.pallas import tpu as pltpu
from jax.experimental.pallas import tpu_sc as plsc

_NC = 2
_NS = 16
_NW = _NC * _NS
_K = 80
_BE = 4000


def _relu(x):
    return jnp.maximum(x, 0.0)


def _wspec(shape):
    return pl.BlockSpec(shape, lambda i: tuple(0 for _ in shape))


def _dot(a, b):
    return jnp.dot(a, b, preferred_element_type=jnp.float32)


def _node_encoder_body(x_ref, w1_ref, b1_ref, w2_ref, b2_ref, o_ref):
    h = _relu(_dot(x_ref[...], w1_ref[...]) + b1_ref[...])
    o_ref[...] = _dot(h, w2_ref[...]) + b2_ref[...]


def _node_encoder(x, p):
    n = x.shape[0]
    do = p['W2'].shape[1]
    return pl.pallas_call(
        _node_encoder_body,
        out_shape=jax.ShapeDtypeStruct((n, do), jnp.float32),
    )(x, p['W1'], p['b1'].reshape(1, -1), p['W2'], p['b2'].reshape(1, -1))


def _node_update_body(ns_ref, a0_ref, a1_ref, w1n_ref, w1a_ref, b1_ref,
                      w2_ref, b2_ref, o_ref):
    agg = a0_ref[...] + a1_ref[...]
    h = _relu(_dot(ns_ref[...], w1n_ref[...]) + _dot(agg, w1a_ref[...])
              + b1_ref[...])
    o_ref[...] = _dot(h, w2_ref[...]) + b2_ref[...]


def _node_update(ns, agg_parts, p):
    n = ns.shape[0]
    d = ns.shape[1]
    w1n = p['W1'][:d]
    w1a = p['W1'][d:]
    do = p['W2'].shape[1]
    return pl.pallas_call(
        _node_update_body,
        out_shape=jax.ShapeDtypeStruct((n, do), jnp.float32),
    )(ns, agg_parts[0], agg_parts[1], w1n, w1a, p['b1'].reshape(1, -1),
      p['W2'], p['b2'].reshape(1, -1))


def _messages_body(ef_ref, sg_ref, dg_ref, ew1_ref, eb1_ref, e2c_ref,
                   wa_ref, wb_ref, bc_ref, w2_ref, b2_ref, o_ref, obf_ref):
    h_e = _relu(_dot(ef_ref[...], ew1_ref[...]) + eb1_ref[...])
    sg = sg_ref[...].astype(jnp.bfloat16)
    dg = dg_ref[...].astype(jnp.bfloat16)
    pre = (_dot(sg, wa_ref[...]) + _dot(dg, wb_ref[...])
           + _dot(h_e, e2c_ref[...]) + bc_ref[...])
    h2 = _relu(pre).astype(jnp.bfloat16)
    m = _dot(h2, w2_ref[...]) + b2_ref[...]
    o_ref[...] = m
    obf_ref[...] = m.astype(jnp.bfloat16)


def _messages(ef, gathered, pe, pm, d_node):
    e = ef.shape[0]
    de = ef.shape[1]
    nblk = e // _BE
    wa = pm['W1'][:d_node].astype(jnp.bfloat16)
    wb = pm['W1'][d_node:2 * d_node].astype(jnp.bfloat16)
    wc = pm['W1'][2 * d_node:]
    e2c = pe['W2'] @ wc
    bc = (pe['b2'] @ wc + pm['b1']).reshape(1, -1)
    w2 = pm['W2'].astype(jnp.bfloat16)
    dh = pm['W1'].shape[1]
    do = pm['W2'].shape[1]
    return pl.pallas_call(
        _messages_body,
        grid=(nblk,),
        in_specs=[
            pl.BlockSpec((_BE, de), lambda i: (i, 0)),
            pl.BlockSpec((_BE, d_node), lambda i: (i, 0)),
            pl.BlockSpec((_BE, d_node), lambda i: (i + nblk, 0)),
            _wspec(pe['W1'].shape),
            _wspec((1, pe['W1'].shape[1])),
            _wspec(e2c.shape),
            _wspec(wa.shape),
            _wspec(wb.shape),
            _wspec((1, dh)),
            _wspec(w2.shape),
            _wspec((1, do)),
        ],
        out_specs=[pl.BlockSpec((_BE, do), lambda i: (i, 0)),
                   pl.BlockSpec((_BE, do), lambda i: (i, 0))],
        out_shape=[jax.ShapeDtypeStruct((e, do), jnp.float32),
                   jax.ShapeDtypeStruct((e, do), jnp.bfloat16)],
    )(ef, gathered, gathered, pe['W1'], pe['b1'].reshape(1, -1), e2c,
      wa, wb, bc, w2, pm['b2'].reshape(1, -1))


def _edge_update_body(ef_ref, usg_ref, udg_ref, m_ref, ew1_ref, eb1_ref,
                      e2c_ref, wa_ref, wb_ref, wd_ref, bc_ref, w2_ref,
                      b2_ref, p1_ref, pb1_ref, p2_ref, pb2_ref,
                      oue_ref, ol_ref):
    h_e = _relu(_dot(ef_ref[...], ew1_ref[...]) + eb1_ref[...])
    usg = usg_ref[...].astype(jnp.bfloat16)
    udg = udg_ref[...].astype(jnp.bfloat16)
    pre = (_dot(usg, wa_ref[...]) + _dot(udg, wb_ref[...])
           + _dot(h_e, e2c_ref[...]) + _dot(m_ref[...], wd_ref[...])
           + bc_ref[...])
    h2 = _relu(pre).astype(jnp.bfloat16)
    ue = _dot(h2, w2_ref[...]) + b2_ref[...]
    oue_ref[...] = ue
    hp = _relu(_dot(ue.astype(jnp.bfloat16), p1_ref[...]) + pb1_ref[...])
    ol_ref[...] = _dot(hp.astype(jnp.bfloat16), p2_ref[...]) + pb2_ref[...]


def _edge_update(ef, gathered, msgs, pe, pu, pp, d_node):
    e = ef.shape[0]
    de = ef.shape[1]
    nblk = e // _BE
    dm = msgs.shape[1]
    wa = pu['W1'][:d_node].astype(jnp.bfloat16)
    wb = pu['W1'][d_node:2 * d_node].astype(jnp.bfloat16)
    wc = pu['W1'][2 * d_node:2 * d_node + pe['W2'].shape[1]]
    wd = pu['W1'][2 * d_node + pe['W2'].shape[1]:].astype(jnp.bfloat16)
    w2u = pu['W2'].astype(jnp.bfloat16)
    p1 = pp['W1'].astype(jnp.bfloat16)
    p2 = pp['W2'].astype(jnp.bfloat16)
    e2c = pe['W2'] @ wc
    bc = (pe['b2'] @ wc + pu['b1']).reshape(1, -1)
    dh = pu['W1'].shape[1]
    do = pu['W2'].shape[1]
    return pl.pallas_call(
        _edge_update_body,
        grid=(nblk,),
        in_specs=[
            pl.BlockSpec((_BE, de), lambda i: (i, 0)),
            pl.BlockSpec((_BE, d_node), lambda i: (i, 0)),
            pl.BlockSpec((_BE, d_node), lambda i: (i + nblk, 0)),
            pl.BlockSpec((_BE, dm), lambda i: (i, 0)),
            _wspec(pe['W1'].shape),
            _wspec((1, pe['W1'].shape[1])),
            _wspec(e2c.shape),
            _wspec(wa.shape),
            _wspec(wb.shape),
            _wspec(wd.shape),
            _wspec((1, dh)),
            _wspec(w2u.shape),
            _wspec((1, do)),
            _wspec(p1.shape),
            _wspec((1, p1.shape[1])),
            _wspec(p2.shape),
            _wspec((1, 1)),
        ],
        out_specs=[
            pl.BlockSpec((_BE, do), lambda i: (i, 0)),
            pl.BlockSpec((_BE, 1), lambda i: (i, 0)),
        ],
        out_shape=[
            jax.ShapeDtypeStruct((e, do), jnp.float32),
            jax.ShapeDtypeStruct((e, 1), jnp.float32),
        ],
    )(ef, gathered, gathered, msgs, pe['W1'], pe['b1'].reshape(1, -1), e2c,
      wa, wb, wd, bc, w2u, pu['b2'].reshape(1, -1),
      p1, pp['b1'].reshape(1, -1), p2, pp['b2'].reshape(1, 1))


_RB = 5
_RBG = 10


def _sc_gather(table, idx):
    b = idx.shape[0]
    d = table.shape[1]
    dt = table.dtype
    b_per_w = b // _NW
    chunks = b_per_w // _K
    groups = chunks // _RBG
    mesh = plsc.VectorSubcoreMesh(core_axis_name="c", subcore_axis_name="s")

    @functools.partial(
        pl.kernel,
        mesh=mesh,
        out_type=jax.ShapeDtypeStruct((b, d), dt),
        scratch_types=(
            [pltpu.VMEM((b_per_w,), jnp.int32)]
            + [pltpu.VMEM((_K, d), dt)] * _RBG
            + [pltpu.SemaphoreType.DMA] * (2 * _RBG)
        ),
    )
    def k(table_hbm, idx_hbm, out_hbm, idx_v, *bufs):
        rows = bufs[:_RBG]
        sg = bufs[_RBG:2 * _RBG]
        sw = bufs[2 * _RBG:]
        wid = lax.axis_index("s") * _NC + lax.axis_index("c")
        base = wid * b_per_w
        pltpu.sync_copy(idx_hbm.at[pl.ds(base, b_per_w)], idx_v)

        def body(g, carry):
            descs = []
            for bb in range(_RBG):
                c = g * _RBG + bb
                off = base + c * _K

                @pl.when(g > 0)
                def _drain_wb(bb=bb, off=off):
                    pltpu.make_async_copy(
                        rows[bb], out_hbm.at[pl.ds(off, _K)], sw[bb]).wait()

                descs.append(pltpu.async_copy(
                    table_hbm.at[idx_v.at[pl.ds(c * _K, _K)]], rows[bb],
                    sg[bb]))
            for bb in range(_RBG):
                c = g * _RBG + bb
                off = base + c * _K
                descs[bb].wait()
                pltpu.async_copy(rows[bb], out_hbm.at[pl.ds(off, _K)],
                                 sw[bb])
            return carry

        lax.fori_loop(0, groups, body, 0)
        for bb in range(_RBG):
            pltpu.make_async_copy(
                rows[bb], out_hbm.at[pl.ds(base, _K)], sw[bb]).wait()

    return k(table, idx)


def _sc_scatter_add(msgs, idx, zeros_init):
    e, d = msgs.shape
    n = zeros_init.shape[0]
    per_core = e // _NC
    per_tile = per_core // _NS
    k_sc = 40
    chunks = per_tile // k_sc
    nz = max(t for t in range(1, _NS + 1) if n % t == 0 and (n // t) % 8 == 0)
    stripe_rows = n // nz
    mesh = plsc.VectorSubcoreMesh(core_axis_name="c", subcore_axis_name="s")

    groups = chunks // _RB

    @functools.partial(
        pl.kernel,
        mesh=mesh,
        out_type=jax.ShapeDtypeStruct((_NC, n, d), jnp.float32),
        scratch_types=(
            [pltpu.VMEM((k_sc,), jnp.int32)] * _RB
            + [pltpu.VMEM((k_sc, d), jnp.float32)] * _RB
            + [pltpu.VMEM_SHARED((n, d), jnp.float32)]
            + [pltpu.SemaphoreType.DMA] * (3 * _RB)
        ),
    )
    def k(msg_hbm, idx_hbm, zero_hbm, out_hbm, *bufs):
        idxs = bufs[:_RB]
        rows = bufs[_RB:2 * _RB]
        acc = bufs[2 * _RB]
        si = bufs[2 * _RB + 1:2 * _RB + 1 + _RB]
        sm = bufs[2 * _RB + 1 + _RB:2 * _RB + 1 + 2 * _RB]
        sa = bufs[2 * _RB + 1 + 2 * _RB:]
        cid = lax.axis_index("c")
        sid = lax.axis_index("s")
        stripe = pl.ds(sid * stripe_rows, stripe_rows)

        @pl.when(sid < nz)
        def _init():
            pltpu.sync_copy(zero_hbm.at[stripe], acc.at[stripe])

        plsc.subcore_barrier()
        base = cid * per_core + sid * per_tile

        def body(g, carry):
            di = []
            dm = []
            for bb in range(_RB):
                c = g * _RB + bb
                off = base + c * k_sc

                @pl.when(g > 0)
                def _drain_add(bb=bb):
                    pltpu.make_async_copy(
                        rows[bb], acc.at[idxs[bb]], sa[bb]).wait()

                di.append(pltpu.async_copy(
                    idx_hbm.at[pl.ds(off, k_sc)], idxs[bb], si[bb]))
                dm.append(pltpu.async_copy(
                    msg_hbm.at[pl.ds(off, k_sc)], rows[bb], sm[bb]))
            for bb in range(_RB):
                di[bb].wait()
                dm[bb].wait()
                pltpu.async_copy(rows[bb], acc.at[idxs[bb]], sa[bb],
                                 add=True)
            return carry

        lax.fori_loop(0, groups, body, 0)
        for bb in range(_RB):
            pltpu.make_async_copy(rows[bb], acc.at[idxs[bb]], sa[bb]).wait()
        plsc.subcore_barrier()

        @pl.when(sid < nz)
        def _writeback():
            pltpu.sync_copy(acc.at[stripe], out_hbm.at[cid].at[stripe])

    return k(msgs, idx, zeros_init)


def kernel(node_features, edge_features, edge_index, num_nodes, params):
    n, d_node = node_features.shape
    e = edge_features.shape[0]

    idx_all = edge_index.reshape(-1).astype(jnp.int32)
    scat_idx = (edge_index[1] % num_nodes).astype(jnp.int32)

    node_state = _node_encoder(node_features, params['node_encoder'])

    gathered = _sc_gather(node_state, idx_all)
    messages, messages_bf = _messages(edge_features, gathered,
                                      params['edge_encoder'],
                                      params['message_encoder'], d_node)

    zeros_init = jnp.zeros((n, messages.shape[1]), jnp.float32)
    agg_parts = _sc_scatter_add(messages, scat_idx, zeros_init)

    updated_node_state = _node_update(node_state, agg_parts,
                                      params['node_updater'])

    gathered2 = _sc_gather(updated_node_state, idx_all)
    updated_edge_state, logits = _edge_update(
        edge_features, gathered2, messages_bf, params['edge_encoder'],
        params['edge_updater'], params['predictor'], d_node)

    return (logits.reshape(e), updated_node_state, updated_edge_state,
            messages)


# speedup vs baseline: 3.3829x; 1.0003x over previous
import functools

import jax
import jax.numpy as jnp
from jax import lax
from jax.experimental import pallas as pl
from jax.experimental.pallas import tpu as pltpu
from jax.experimental.pallas import tpu_sc as plsc

_NC = 2
_NS = 16
_NW = _NC * _NS
_K = 80
_BE = 4000


def _relu(x):
    return jnp.maximum(x, 0.0)


def _wspec(shape):
    return pl.BlockSpec(shape, lambda i: tuple(0 for _ in shape))


def _dot(a, b):
    return jnp.dot(a, b, preferred_element_type=jnp.float32)


def _node_encoder_body(x_ref, w1_ref, b1_ref, w2_ref, b2_ref, o_ref):
    h = _relu(_dot(x_ref[...], w1_ref[...]) + b1_ref[...])
    o_ref[...] = _dot(h, w2_ref[...]) + b2_ref[...]


def _node_encoder(x, p):
    n = x.shape[0]
    do = p['W2'].shape[1]
    return pl.pallas_call(
        _node_encoder_body,
        out_shape=jax.ShapeDtypeStruct((n, do), jnp.float32),
    )(x, p['W1'], p['b1'].reshape(1, -1), p['W2'], p['b2'].reshape(1, -1))


def _node_update_body(ns_ref, a0_ref, a1_ref, w1n_ref, w1a_ref, b1_ref,
                      w2_ref, b2_ref, o_ref):
    agg = a0_ref[...] + a1_ref[...]
    h = _relu(_dot(ns_ref[...], w1n_ref[...]) + _dot(agg, w1a_ref[...])
              + b1_ref[...])
    o_ref[...] = _dot(h, w2_ref[...]) + b2_ref[...]


def _node_update(ns, agg_parts, p):
    n = ns.shape[0]
    d = ns.shape[1]
    w1n = p['W1'][:d]
    w1a = p['W1'][d:]
    do = p['W2'].shape[1]
    return pl.pallas_call(
        _node_update_body,
        out_shape=jax.ShapeDtypeStruct((n, do), jnp.float32),
    )(ns, agg_parts[0], agg_parts[1], w1n, w1a, p['b1'].reshape(1, -1),
      p['W2'], p['b2'].reshape(1, -1))


def _messages_body(ef_ref, sg_ref, dg_ref, ew1_ref, eb1_ref, e2c_ref,
                   wa_ref, wb_ref, bc_ref, w2_ref, b2_ref, o_ref, obf_ref):
    h_e = _relu(_dot(ef_ref[...], ew1_ref[...]) + eb1_ref[...])
    sg = sg_ref[...].astype(jnp.bfloat16)
    dg = dg_ref[...].astype(jnp.bfloat16)
    pre = (_dot(sg, wa_ref[...]) + _dot(dg, wb_ref[...])
           + _dot(h_e, e2c_ref[...]) + bc_ref[...])
    h2 = _relu(pre).astype(jnp.bfloat16)
    m = _dot(h2, w2_ref[...]) + b2_ref[...]
    o_ref[...] = m
    obf_ref[...] = m.astype(jnp.bfloat16)


def _alias_body(fn):
    def wrapped(*refs):
        n_extra = 2
        args = refs[:-(2 + n_extra)] + refs[-2:]
        fn(*args)
    return wrapped


def _messages(ef, gathered, pe, pm, d_node, h, prev):
    e = ef.shape[0]
    de = ef.shape[1]
    e_half = gathered.shape[0] // 2
    nblk = e_half // _BE
    wa = pm['W1'][:d_node].astype(jnp.bfloat16)
    wb = pm['W1'][d_node:2 * d_node].astype(jnp.bfloat16)
    wc = pm['W1'][2 * d_node:]
    e2c = pe['W2'] @ wc
    bc = (pe['b2'] @ wc + pm['b1']).reshape(1, -1)
    w2 = pm['W2'].astype(jnp.bfloat16)
    dh = pm['W1'].shape[1]
    do = pm['W2'].shape[1]
    in_specs = [
        pl.BlockSpec((_BE, de), lambda i: (i + h * nblk, 0)),
        pl.BlockSpec((_BE, d_node), lambda i: (i, 0)),
        pl.BlockSpec((_BE, d_node), lambda i: (i + nblk, 0)),
        _wspec(pe['W1'].shape),
        _wspec((1, pe['W1'].shape[1])),
        _wspec(e2c.shape),
        _wspec(wa.shape),
        _wspec(wb.shape),
        _wspec((1, dh)),
        _wspec(w2.shape),
        _wspec((1, do)),
    ]
    args = [ef, gathered, gathered, pe['W1'], pe['b1'].reshape(1, -1), e2c,
            wa, wb, bc, w2, pm['b2'].reshape(1, -1)]
    body = _messages_body
    aliases = {}
    if prev is not None:
        in_specs = in_specs + [
            pl.BlockSpec((8, do), lambda i: (0, 0)),
            pl.BlockSpec((16, do), lambda i: (0, 0)),
        ]
        args = args + [prev[0], prev[1]]
        aliases = {len(args) - 2: 0, len(args) - 1: 1}
        body = _alias_body(_messages_body)
    return pl.pallas_call(
        body,
        grid=(nblk,),
        in_specs=in_specs,
        out_specs=[pl.BlockSpec((_BE, do), lambda i: (i + h * nblk, 0)),
                   pl.BlockSpec((_BE, do), lambda i: (i + h * nblk, 0))],
        out_shape=[jax.ShapeDtypeStruct((e, do), jnp.float32),
                   jax.ShapeDtypeStruct((e, do), jnp.bfloat16)],
        input_output_aliases=aliases,
    )(*args)


def _edge_update_body(ef_ref, usg_ref, udg_ref, m_ref, ew1_ref, eb1_ref,
                      e2c_ref, wa_ref, wb_ref, wd_ref, bc_ref, w2_ref,
                      b2_ref, p1_ref, pb1_ref, p2_ref, pb2_ref,
                      oue_ref, ol_ref):
    h_e = _relu(_dot(ef_ref[...], ew1_ref[...]) + eb1_ref[...])
    usg = usg_ref[...].astype(jnp.bfloat16)
    udg = udg_ref[...].astype(jnp.bfloat16)
    pre = (_dot(usg, wa_ref[...]) + _dot(udg, wb_ref[...])
           + _dot(h_e, e2c_ref[...]) + _dot(m_ref[...], wd_ref[...])
           + bc_ref[...])
    h2 = _relu(pre).astype(jnp.bfloat16)
    ue = _dot(h2, w2_ref[...]) + b2_ref[...]
    oue_ref[...] = ue
    hp = _relu(_dot(ue.astype(jnp.bfloat16), p1_ref[...]) + pb1_ref[...])
    ol_ref[...] = _dot(hp.astype(jnp.bfloat16), p2_ref[...]) + pb2_ref[...]


def _edge_update(ef, gathered, msgs, pe, pu, pp, d_node, h, prev):
    e = ef.shape[0]
    de = ef.shape[1]
    e_half = gathered.shape[0] // 2
    nblk = e_half // _BE
    dm = msgs.shape[1]
    wa = pu['W1'][:d_node].astype(jnp.bfloat16)
    wb = pu['W1'][d_node:2 * d_node].astype(jnp.bfloat16)
    wc = pu['W1'][2 * d_node:2 * d_node + pe['W2'].shape[1]]
    wd = pu['W1'][2 * d_node + pe['W2'].shape[1]:].astype(jnp.bfloat16)
    w2u = pu['W2'].astype(jnp.bfloat16)
    p1 = pp['W1'].astype(jnp.bfloat16)
    p2 = pp['W2'].astype(jnp.bfloat16)
    e2c = pe['W2'] @ wc
    bc = (pe['b2'] @ wc + pu['b1']).reshape(1, -1)
    dh = pu['W1'].shape[1]
    do = pu['W2'].shape[1]
    in_specs = [
        pl.BlockSpec((_BE, de), lambda i: (i + h * nblk, 0)),
        pl.BlockSpec((_BE, d_node), lambda i: (i, 0)),
        pl.BlockSpec((_BE, d_node), lambda i: (i + nblk, 0)),
        pl.BlockSpec((_BE, dm), lambda i: (i + h * nblk, 0)),
        _wspec(pe['W1'].shape),
        _wspec((1, pe['W1'].shape[1])),
        _wspec(e2c.shape),
        _wspec(wa.shape),
        _wspec(wb.shape),
        _wspec(wd.shape),
        _wspec((1, dh)),
        _wspec(w2u.shape),
        _wspec((1, do)),
        _wspec(p1.shape),
        _wspec((1, p1.shape[1])),
        _wspec(p2.shape),
        _wspec((1, 1)),
    ]
    args = [ef, gathered, gathered, msgs, pe['W1'],
            pe['b1'].reshape(1, -1), e2c, wa, wb, wd, bc, w2u,
            pu['b2'].reshape(1, -1), p1, pp['b1'].reshape(1, -1), p2,
            pp['b2'].reshape(1, 1)]
    body = _edge_update_body
    aliases = {}
    if prev is not None:
        in_specs = in_specs + [
            pl.BlockSpec((8, do), lambda i: (0, 0)),
            pl.BlockSpec((8, 1), lambda i: (0, 0)),
        ]
        args = args + [prev[0], prev[1]]
        aliases = {len(args) - 2: 0, len(args) - 1: 1}
        body = _alias_body(_edge_update_body)
    return pl.pallas_call(
        body,
        grid=(nblk,),
        in_specs=in_specs,
        out_specs=[
            pl.BlockSpec((_BE, do), lambda i: (i + h * nblk, 0)),
            pl.BlockSpec((_BE, 1), lambda i: (i + h * nblk, 0)),
        ],
        out_shape=[
            jax.ShapeDtypeStruct((e, do), jnp.float32),
            jax.ShapeDtypeStruct((e, 1), jnp.float32),
        ],
        input_output_aliases=aliases,
    )(*args)


_RB = 5
_RBG = 5


def _sc_gather_half(table, idx, h, e_half):
    e_total = idx.shape[0] // 2
    d = table.shape[1]
    dt = table.dtype
    b_out = 2 * e_half
    b_per_w = b_out // _NW
    chunks = b_per_w // _K
    groups = chunks // _RBG
    mesh = plsc.VectorSubcoreMesh(core_axis_name="c", subcore_axis_name="s")

    @functools.partial(
        pl.kernel,
        mesh=mesh,
        out_type=jax.ShapeDtypeStruct((b_out, d), dt),
        scratch_types=(
            [pltpu.VMEM((b_per_w,), jnp.int32)]
            + [pltpu.VMEM((_K, d), dt)] * _RBG
            + [pltpu.SemaphoreType.DMA] * (2 * _RBG)
        ),
    )
    def k(table_hbm, idx_hbm, out_hbm, idx_v, *bufs):
        rows = bufs[:_RBG]
        sg = bufs[_RBG:2 * _RBG]
        sw = bufs[2 * _RBG:]
        wid = lax.axis_index("s") * _NC + lax.axis_index("c")
        part = wid // _NS
        lane = wid % _NS
        idx_base = h * e_half + part * e_total + lane * b_per_w
        out_base = part * e_half + lane * b_per_w
        pltpu.sync_copy(idx_hbm.at[pl.ds(idx_base, b_per_w)], idx_v)

        def body(g, carry):
            descs = []
            for bb in range(_RBG):
                c = g * _RBG + bb
                off = out_base + c * _K

                @pl.when(g > 0)
                def _drain_wb(bb=bb, off=off):
                    pltpu.make_async_copy(
                        rows[bb], out_hbm.at[pl.ds(off, _K)], sw[bb]).wait()

                descs.append(pltpu.async_copy(
                    table_hbm.at[idx_v.at[pl.ds(c * _K, _K)]], rows[bb],
                    sg[bb]))
            for bb in range(_RBG):
                c = g * _RBG + bb
                off = out_base + c * _K
                descs[bb].wait()
                pltpu.async_copy(rows[bb], out_hbm.at[pl.ds(off, _K)],
                                 sw[bb])
            return carry

        lax.fori_loop(0, groups, body, 0)
        for bb in range(_RBG):
            pltpu.make_async_copy(
                rows[bb], out_hbm.at[pl.ds(out_base, _K)], sw[bb]).wait()

    return k(table, idx)


def _sc_scatter_add(msgs, idx, zeros_init):
    e, d = msgs.shape
    n = zeros_init.shape[0]
    per_core = e // _NC
    per_tile = per_core // _NS
    k_sc = 40
    chunks = per_tile // k_sc
    nz = max(t for t in range(1, _NS + 1) if n % t == 0 and (n // t) % 8 == 0)
    stripe_rows = n // nz
    mesh = plsc.VectorSubcoreMesh(core_axis_name="c", subcore_axis_name="s")

    groups = chunks // _RB

    @functools.partial(
        pl.kernel,
        mesh=mesh,
        out_type=jax.ShapeDtypeStruct((_NC, n, d), jnp.float32),
        scratch_types=(
            [pltpu.VMEM((k_sc,), jnp.int32)] * _RB
            + [pltpu.VMEM((k_sc, d), jnp.float32)] * _RB
            + [pltpu.VMEM_SHARED((n, d), jnp.float32)]
            + [pltpu.SemaphoreType.DMA] * (3 * _RB)
        ),
    )
    def k(msg_hbm, idx_hbm, zero_hbm, out_hbm, *bufs):
        idxs = bufs[:_RB]
        rows = bufs[_RB:2 * _RB]
        acc = bufs[2 * _RB]
        si = bufs[2 * _RB + 1:2 * _RB + 1 + _RB]
        sm = bufs[2 * _RB + 1 + _RB:2 * _RB + 1 + 2 * _RB]
        sa = bufs[2 * _RB + 1 + 2 * _RB:]
        cid = lax.axis_index("c")
        sid = lax.axis_index("s")
        stripe = pl.ds(sid * stripe_rows, stripe_rows)

        @pl.when(sid < nz)
        def _init():
            pltpu.sync_copy(zero_hbm.at[stripe], acc.at[stripe])

        plsc.subcore_barrier()
        base = cid * per_core + sid * per_tile

        def body(g, carry):
            di = []
            dm = []
            for bb in range(_RB):
                c = g * _RB + bb
                off = base + c * k_sc

                @pl.when(g > 0)
                def _drain_add(bb=bb):
                    pltpu.make_async_copy(
                        rows[bb], acc.at[idxs[bb]], sa[bb]).wait()

                di.append(pltpu.async_copy(
                    idx_hbm.at[pl.ds(off, k_sc)], idxs[bb], si[bb]))
                dm.append(pltpu.async_copy(
                    msg_hbm.at[pl.ds(off, k_sc)], rows[bb], sm[bb]))
            for bb in range(_RB):
                di[bb].wait()
                dm[bb].wait()
                pltpu.async_copy(rows[bb], acc.at[idxs[bb]], sa[bb],
                                 add=True)
            return carry

        lax.fori_loop(0, groups, body, 0)
        for bb in range(_RB):
            pltpu.make_async_copy(rows[bb], acc.at[idxs[bb]], sa[bb]).wait()
        plsc.subcore_barrier()

        @pl.when(sid < nz)
        def _writeback():
            pltpu.sync_copy(acc.at[stripe], out_hbm.at[cid].at[stripe])

    return k(msgs, idx, zeros_init)


def kernel(node_features, edge_features, edge_index, num_nodes, params):
    n, d_node = node_features.shape
    e = edge_features.shape[0]

    idx_all = edge_index.reshape(-1).astype(jnp.int32)
    scat_idx = (edge_index[1] % num_nodes).astype(jnp.int32)

    e_half = e // 2
    node_state = _node_encoder(node_features, params['node_encoder'])

    g0 = _sc_gather_half(node_state, idx_all, 0, e_half)
    g1 = _sc_gather_half(node_state, idx_all, 1, e_half)
    m0 = _messages(edge_features, g0, params['edge_encoder'],
                   params['message_encoder'], d_node, 0, None)
    messages, messages_bf = _messages(edge_features, g1,
                                      params['edge_encoder'],
                                      params['message_encoder'], d_node,
                                      1, m0)

    zeros_init = jnp.zeros((n, messages.shape[1]), jnp.float32)
    agg_parts = _sc_scatter_add(messages, scat_idx, zeros_init)

    updated_node_state = _node_update(node_state, agg_parts,
                                      params['node_updater'])

    g2 = _sc_gather_half(updated_node_state, idx_all, 0, e_half)
    g3 = _sc_gather_half(updated_node_state, idx_all, 1, e_half)
    u0 = _edge_update(edge_features, g2, messages_bf,
                      params['edge_encoder'], params['edge_updater'],
                      params['predictor'], d_node, 0, None)
    updated_edge_state, logits = _edge_update(
        edge_features, g3, messages_bf, params['edge_encoder'],
        params['edge_updater'], params['predictor'], d_node, 1, u0)

    return (logits.reshape(e), updated_node_state, updated_edge_state,
            messages)


# speedup vs baseline: 3.3877x; 1.0014x over previous
import functools

import jax
import jax.numpy as jnp
from jax import lax
from jax.experimental import pallas as pl
from jax.experimental.pallas import tpu as pltpu
from jax.experimental.pallas import tpu_sc as plsc

_NC = 2
_NS = 16
_NW = _NC * _NS
_K = 80
_BE = 4000


def _relu(x):
    return jnp.maximum(x, 0.0)


def _wspec(shape):
    return pl.BlockSpec(shape, lambda i: tuple(0 for _ in shape))


def _dot(a, b):
    return jnp.dot(a, b, preferred_element_type=jnp.float32)


def _node_encoder_body(x_ref, w1_ref, b1_ref, w2_ref, b2_ref, o_ref):
    h = _relu(_dot(x_ref[...], w1_ref[...]) + b1_ref[...])
    o_ref[...] = _dot(h, w2_ref[...]) + b2_ref[...]


def _node_encoder(x, p):
    n = x.shape[0]
    do = p['W2'].shape[1]
    return pl.pallas_call(
        _node_encoder_body,
        out_shape=jax.ShapeDtypeStruct((n, do), jnp.float32),
    )(x, p['W1'], p['b1'].reshape(1, -1), p['W2'], p['b2'].reshape(1, -1))


def _node_update_body(ns_ref, a0_ref, a1_ref, w1n_ref, w1a_ref, b1_ref,
                      w2_ref, b2_ref, o_ref):
    agg = a0_ref[...] + a1_ref[...]
    h = _relu(_dot(ns_ref[...], w1n_ref[...]) + _dot(agg, w1a_ref[...])
              + b1_ref[...])
    o_ref[...] = _dot(h, w2_ref[...]) + b2_ref[...]


def _node_update(ns, agg_parts, p):
    n = ns.shape[0]
    d = ns.shape[1]
    w1n = p['W1'][:d]
    w1a = p['W1'][d:]
    do = p['W2'].shape[1]
    return pl.pallas_call(
        _node_update_body,
        out_shape=jax.ShapeDtypeStruct((n, do), jnp.float32),
    )(ns, agg_parts[0], agg_parts[1], w1n, w1a, p['b1'].reshape(1, -1),
      p['W2'], p['b2'].reshape(1, -1))


def _messages_body(ef_ref, sg_ref, dg_ref, ew1_ref, eb1_ref, e2c_ref,
                   wa_ref, wb_ref, bc_ref, w2_ref, b2_ref, o_ref, obf_ref):
    h_e = _relu(_dot(ef_ref[...], ew1_ref[...]) + eb1_ref[...])
    sg = sg_ref[...].astype(jnp.bfloat16)
    dg = dg_ref[...].astype(jnp.bfloat16)
    pre = (_dot(sg, wa_ref[...]) + _dot(dg, wb_ref[...])
           + _dot(h_e, e2c_ref[...]) + bc_ref[...])
    h2 = _relu(pre).astype(jnp.bfloat16)
    m = _dot(h2, w2_ref[...]) + b2_ref[...]
    o_ref[...] = m
    obf_ref[...] = m.astype(jnp.bfloat16)


def _messages(ef, gathered, pe, pm, d_node):
    e = ef.shape[0]
    de = ef.shape[1]
    nblk = e // _BE
    wa = pm['W1'][:d_node].astype(jnp.bfloat16)
    wb = pm['W1'][d_node:2 * d_node].astype(jnp.bfloat16)
    wc = pm['W1'][2 * d_node:]
    e2c = pe['W2'] @ wc
    bc = (pe['b2'] @ wc + pm['b1']).reshape(1, -1)
    w2 = pm['W2'].astype(jnp.bfloat16)
    dh = pm['W1'].shape[1]
    do = pm['W2'].shape[1]
    return pl.pallas_call(
        _messages_body,
        grid=(nblk,),
        in_specs=[
            pl.BlockSpec((_BE, de), lambda i: (i, 0)),
            pl.BlockSpec((_BE, d_node), lambda i: (i, 0)),
            pl.BlockSpec((_BE, d_node), lambda i: (i + nblk, 0)),
            _wspec(pe['W1'].shape),
            _wspec((1, pe['W1'].shape[1])),
            _wspec(e2c.shape),
            _wspec(wa.shape),
            _wspec(wb.shape),
            _wspec((1, dh)),
            _wspec(w2.shape),
            _wspec((1, do)),
        ],
        out_specs=[pl.BlockSpec((_BE, do), lambda i: (i, 0)),
                   pl.BlockSpec((_BE, do), lambda i: (i, 0))],
        out_shape=[jax.ShapeDtypeStruct((e, do), jnp.float32),
                   jax.ShapeDtypeStruct((e, do), jnp.bfloat16)],
    )(ef, gathered, gathered, pe['W1'], pe['b1'].reshape(1, -1), e2c,
      wa, wb, bc, w2, pm['b2'].reshape(1, -1))


def _edge_update_body(ef_ref, usg_ref, udg_ref, m_ref, ew1_ref, eb1_ref,
                      e2c_ref, wa_ref, wb_ref, wd_ref, bc_ref, w2_ref,
                      b2_ref, p1_ref, pb1_ref, p2_ref, pb2_ref,
                      oue_ref, ol_ref):
    h_e = _relu(_dot(ef_ref[...], ew1_ref[...]) + eb1_ref[...])
    usg = usg_ref[...].astype(jnp.bfloat16)
    udg = udg_ref[...].astype(jnp.bfloat16)
    pre = (_dot(usg, wa_ref[...]) + _dot(udg, wb_ref[...])
           + _dot(h_e, e2c_ref[...]) + _dot(m_ref[...], wd_ref[...])
           + bc_ref[...])
    h2 = _relu(pre).astype(jnp.bfloat16)
    ue = _dot(h2, w2_ref[...]) + b2_ref[...]
    oue_ref[...] = ue
    hp = _relu(_dot(ue.astype(jnp.bfloat16), p1_ref[...]) + pb1_ref[...])
    ol_ref[...] = _dot(hp.astype(jnp.bfloat16), p2_ref[...]) + pb2_ref[...]


def _edge_update(ef, gathered, msgs, pe, pu, pp, d_node):
    e = ef.shape[0]
    de = ef.shape[1]
    nblk = e // _BE
    dm = msgs.shape[1]
    wa = pu['W1'][:d_node].astype(jnp.bfloat16)
    wb = pu['W1'][d_node:2 * d_node].astype(jnp.bfloat16)
    wc = pu['W1'][2 * d_node:2 * d_node + pe['W2'].shape[1]]
    wd = pu['W1'][2 * d_node + pe['W2'].shape[1]:].astype(jnp.bfloat16)
    w2u = pu['W2'].astype(jnp.bfloat16)
    p1 = pp['W1'].astype(jnp.bfloat16)
    p2 = pp['W2'].astype(jnp.bfloat16)
    e2c = pe['W2'] @ wc
    bc = (pe['b2'] @ wc + pu['b1']).reshape(1, -1)
    dh = pu['W1'].shape[1]
    do = pu['W2'].shape[1]
    return pl.pallas_call(
        _edge_update_body,
        grid=(nblk,),
        in_specs=[
            pl.BlockSpec((_BE, de), lambda i: (i, 0)),
            pl.BlockSpec((_BE, d_node), lambda i: (i, 0)),
            pl.BlockSpec((_BE, d_node), lambda i: (i + nblk, 0)),
            pl.BlockSpec((_BE, dm), lambda i: (i, 0)),
            _wspec(pe['W1'].shape),
            _wspec((1, pe['W1'].shape[1])),
            _wspec(e2c.shape),
            _wspec(wa.shape),
            _wspec(wb.shape),
            _wspec(wd.shape),
            _wspec((1, dh)),
            _wspec(w2u.shape),
            _wspec((1, do)),
            _wspec(p1.shape),
            _wspec((1, p1.shape[1])),
            _wspec(p2.shape),
            _wspec((1, 1)),
        ],
        out_specs=[
            pl.BlockSpec((_BE, do), lambda i: (i, 0)),
            pl.BlockSpec((_BE, 1), lambda i: (i, 0)),
        ],
        out_shape=[
            jax.ShapeDtypeStruct((e, do), jnp.float32),
            jax.ShapeDtypeStruct((e, 1), jnp.float32),
        ],
    )(ef, gathered, gathered, msgs, pe['W1'], pe['b1'].reshape(1, -1), e2c,
      wa, wb, wd, bc, w2u, pu['b2'].reshape(1, -1),
      p1, pp['b1'].reshape(1, -1), p2, pp['b2'].reshape(1, 1))


_RB = 5
_RBG = 10


def _sc_gather(table, idx):
    b = idx.shape[0]
    d = table.shape[1]
    dt = table.dtype
    b_per_w = b // _NW
    chunks = b_per_w // _K
    groups = chunks // _RBG
    mesh = plsc.VectorSubcoreMesh(core_axis_name="c", subcore_axis_name="s")

    @functools.partial(
        pl.kernel,
        mesh=mesh,
        out_type=jax.ShapeDtypeStruct((b, d), dt),
        scratch_types=(
            [pltpu.VMEM((b_per_w,), jnp.int32)]
            + [pltpu.VMEM((_K, d), dt)] * _RBG
            + [pltpu.SemaphoreType.DMA] * (2 * _RBG)
        ),
    )
    def k(table_hbm, idx_hbm, out_hbm, idx_v, *bufs):
        rows = bufs[:_RBG]
        sg = bufs[_RBG:2 * _RBG]
        sw = bufs[2 * _RBG:]
        wid = lax.axis_index("s") * _NC + lax.axis_index("c")
        base = wid * b_per_w
        pltpu.sync_copy(idx_hbm.at[pl.ds(base, b_per_w)], idx_v)

        def body(g, carry):
            descs = []
            for bb in range(_RBG):
                c = g * _RBG + bb
                off = base + c * _K

                @pl.when(g > 0)
                def _drain_wb(bb=bb, off=off):
                    pltpu.make_async_copy(
                        rows[bb], out_hbm.at[pl.ds(off, _K)], sw[bb]).wait()

                descs.append(pltpu.async_copy(
                    table_hbm.at[idx_v.at[pl.ds(c * _K, _K)]], rows[bb],
                    sg[bb]))
            for bb in range(_RBG):
                c = g * _RBG + bb
                off = base + c * _K
                descs[bb].wait()
                pltpu.async_copy(rows[bb], out_hbm.at[pl.ds(off, _K)],
                                 sw[bb])
            return carry

        lax.fori_loop(0, groups, body, 0)
        for bb in range(_RBG):
            pltpu.make_async_copy(
                rows[bb], out_hbm.at[pl.ds(base, _K)], sw[bb]).wait()

    return k(table, idx)


def _sc_scatter_add(msgs, idx, n):
    e, d = msgs.shape
    per_core = e // _NC
    per_tile = per_core // _NS
    k_sc = 40
    chunks = per_tile // k_sc
    nz = max(t for t in range(1, _NS + 1) if n % t == 0 and (n // t) % 8 == 0)
    stripe_rows = n // nz
    zeros_init = jnp.zeros((stripe_rows, d), jnp.float32)
    mesh = plsc.VectorSubcoreMesh(core_axis_name="c", subcore_axis_name="s")

    groups = chunks // _RB

    @functools.partial(
        pl.kernel,
        mesh=mesh,
        out_type=jax.ShapeDtypeStruct((_NC, n, d), jnp.float32),
        scratch_types=(
            [pltpu.VMEM((k_sc,), jnp.int32)] * _RB
            + [pltpu.VMEM((k_sc, d), jnp.float32)] * _RB
            + [pltpu.VMEM_SHARED((n, d), jnp.float32)]
            + [pltpu.SemaphoreType.DMA] * (3 * _RB)
        ),
    )
    def k(msg_hbm, idx_hbm, zero_hbm, out_hbm, *bufs):
        idxs = bufs[:_RB]
        rows = bufs[_RB:2 * _RB]
        acc = bufs[2 * _RB]
        si = bufs[2 * _RB + 1:2 * _RB + 1 + _RB]
        sm = bufs[2 * _RB + 1 + _RB:2 * _RB + 1 + 2 * _RB]
        sa = bufs[2 * _RB + 1 + 2 * _RB:]
        cid = lax.axis_index("c")
        sid = lax.axis_index("s")
        stripe = pl.ds(sid * stripe_rows, stripe_rows)

        @pl.when(sid < nz)
        def _init():
            pltpu.sync_copy(zero_hbm, acc.at[stripe])

        plsc.subcore_barrier()
        base = cid * per_core + sid * per_tile

        def body(g, carry):
            di = []
            dm = []
            for bb in range(_RB):
                c = g * _RB + bb
                off = base + c * k_sc

                @pl.when(g > 0)
                def _drain_add(bb=bb):
                    pltpu.make_async_copy(
                        rows[bb], acc.at[idxs[bb]], sa[bb]).wait()

                di.append(pltpu.async_copy(
                    idx_hbm.at[pl.ds(off, k_sc)], idxs[bb], si[bb]))
                dm.append(pltpu.async_copy(
                    msg_hbm.at[pl.ds(off, k_sc)], rows[bb], sm[bb]))
            for bb in range(_RB):
                di[bb].wait()
                dm[bb].wait()
                pltpu.async_copy(rows[bb], acc.at[idxs[bb]], sa[bb],
                                 add=True)
            return carry

        lax.fori_loop(0, groups, body, 0)
        for bb in range(_RB):
            pltpu.make_async_copy(rows[bb], acc.at[idxs[bb]], sa[bb]).wait()
        plsc.subcore_barrier()

        @pl.when(sid < nz)
        def _writeback():
            pltpu.sync_copy(acc.at[stripe], out_hbm.at[cid].at[stripe])

    return k(msgs, idx, zeros_init)


def kernel(node_features, edge_features, edge_index, num_nodes, params):
    n, d_node = node_features.shape
    e = edge_features.shape[0]

    idx_all = edge_index.reshape(-1).astype(jnp.int32)
    scat_idx = (edge_index[1] % num_nodes).astype(jnp.int32)

    node_state = _node_encoder(node_features, params['node_encoder'])

    gathered = _sc_gather(node_state, idx_all)
    messages, messages_bf = _messages(edge_features, gathered,
                                      params['edge_encoder'],
                                      params['message_encoder'], d_node)

    agg_parts = _sc_scatter_add(messages, scat_idx, n)

    updated_node_state = _node_update(node_state, agg_parts,
                                      params['node_updater'])

    gathered2 = _sc_gather(updated_node_state, idx_all)
    updated_edge_state, logits = _edge_update(
        edge_features, gathered2, messages_bf, params['edge_encoder'],
        params['edge_updater'], params['predictor'], d_node)

    return (logits.reshape(e), updated_node_state, updated_edge_state,
            messages)


# speedup vs baseline: 3.4151x; 1.0081x over previous
import functools

import jax
import jax.numpy as jnp
from jax import lax
from jax.experimental import pallas as pl
from jax.experimental.pallas import tpu as pltpu
from jax.experimental.pallas import tpu_sc as plsc

_NC = 2
_NS = 16
_NW = _NC * _NS
_K = 80
_BE = 8000


def _relu(x):
    return jnp.maximum(x, 0.0)


def _wspec(shape):
    return pl.BlockSpec(shape, lambda i: tuple(0 for _ in shape))


def _dot(a, b):
    return jnp.dot(a, b, preferred_element_type=jnp.float32)


def _node_encoder_body(x_ref, w1_ref, b1_ref, w2_ref, b2_ref, o_ref):
    h = _relu(_dot(x_ref[...], w1_ref[...]) + b1_ref[...])
    o_ref[...] = _dot(h, w2_ref[...]) + b2_ref[...]


def _node_encoder(x, p):
    n = x.shape[0]
    do = p['W2'].shape[1]
    return pl.pallas_call(
        _node_encoder_body,
        out_shape=jax.ShapeDtypeStruct((n, do), jnp.float32),
    )(x, p['W1'], p['b1'].reshape(1, -1), p['W2'], p['b2'].reshape(1, -1))


def _node_update_body(ns_ref, a0_ref, a1_ref, w1n_ref, w1a_ref, b1_ref,
                      w2_ref, b2_ref, o_ref):
    agg = a0_ref[...] + a1_ref[...]
    h = _relu(_dot(ns_ref[...], w1n_ref[...]) + _dot(agg, w1a_ref[...])
              + b1_ref[...])
    o_ref[...] = _dot(h, w2_ref[...]) + b2_ref[...]


def _node_update(ns, agg_parts, p):
    n = ns.shape[0]
    d = ns.shape[1]
    w1n = p['W1'][:d]
    w1a = p['W1'][d:]
    do = p['W2'].shape[1]
    return pl.pallas_call(
        _node_update_body,
        out_shape=jax.ShapeDtypeStruct((n, do), jnp.float32),
    )(ns, agg_parts[0], agg_parts[1], w1n, w1a, p['b1'].reshape(1, -1),
      p['W2'], p['b2'].reshape(1, -1))


def _messages_body(ef_ref, sg_ref, dg_ref, ew1_ref, eb1_ref, e2c_ref,
                   wa_ref, wb_ref, bc_ref, w2_ref, b2_ref, o_ref, obf_ref):
    h_e = _relu(_dot(ef_ref[...], ew1_ref[...]) + eb1_ref[...])
    sg = sg_ref[...].astype(jnp.bfloat16)
    dg = dg_ref[...].astype(jnp.bfloat16)
    pre = (_dot(sg, wa_ref[...]) + _dot(dg, wb_ref[...])
           + _dot(h_e, e2c_ref[...]) + bc_ref[...])
    h2 = _relu(pre).astype(jnp.bfloat16)
    m = _dot(h2, w2_ref[...]) + b2_ref[...]
    o_ref[...] = m
    obf_ref[...] = m.astype(jnp.bfloat16)


def _messages(ef, gathered, pe, pm, d_node):
    e = ef.shape[0]
    de = ef.shape[1]
    nblk = e // _BE
    wa = pm['W1'][:d_node].astype(jnp.bfloat16)
    wb = pm['W1'][d_node:2 * d_node].astype(jnp.bfloat16)
    wc = pm['W1'][2 * d_node:]
    e2c = pe['W2'] @ wc
    bc = (pe['b2'] @ wc + pm['b1']).reshape(1, -1)
    w2 = pm['W2'].astype(jnp.bfloat16)
    dh = pm['W1'].shape[1]
    do = pm['W2'].shape[1]
    return pl.pallas_call(
        _messages_body,
        grid=(nblk,),
        in_specs=[
            pl.BlockSpec((_BE, de), lambda i: (i, 0)),
            pl.BlockSpec((_BE, d_node), lambda i: (i, 0)),
            pl.BlockSpec((_BE, d_node), lambda i: (i + nblk, 0)),
            _wspec(pe['W1'].shape),
            _wspec((1, pe['W1'].shape[1])),
            _wspec(e2c.shape),
            _wspec(wa.shape),
            _wspec(wb.shape),
            _wspec((1, dh)),
            _wspec(w2.shape),
            _wspec((1, do)),
        ],
        out_specs=[pl.BlockSpec((_BE, do), lambda i: (i, 0)),
                   pl.BlockSpec((_BE, do), lambda i: (i, 0))],
        out_shape=[jax.ShapeDtypeStruct((e, do), jnp.float32),
                   jax.ShapeDtypeStruct((e, do), jnp.bfloat16)],
    )(ef, gathered, gathered, pe['W1'], pe['b1'].reshape(1, -1), e2c,
      wa, wb, bc, w2, pm['b2'].reshape(1, -1))


def _edge_update_body(ef_ref, usg_ref, udg_ref, m_ref, ew1_ref, eb1_ref,
                      e2c_ref, wa_ref, wb_ref, wd_ref, bc_ref, w2_ref,
                      b2_ref, p1_ref, pb1_ref, p2_ref, pb2_ref,
                      oue_ref, ol_ref):
    h_e = _relu(_dot(ef_ref[...], ew1_ref[...]) + eb1_ref[...])
    usg = usg_ref[...].astype(jnp.bfloat16)
    udg = udg_ref[...].astype(jnp.bfloat16)
    pre = (_dot(usg, wa_ref[...]) + _dot(udg, wb_ref[...])
           + _dot(h_e, e2c_ref[...]) + _dot(m_ref[...], wd_ref[...])
           + bc_ref[...])
    h2 = _relu(pre).astype(jnp.bfloat16)
    ue = _dot(h2, w2_ref[...]) + b2_ref[...]
    oue_ref[...] = ue
    hp = _relu(_dot(ue.astype(jnp.bfloat16), p1_ref[...]) + pb1_ref[...])
    ol_ref[...] = _dot(hp.astype(jnp.bfloat16), p2_ref[...]) + pb2_ref[...]


def _edge_update(ef, gathered, msgs, pe, pu, pp, d_node):
    e = ef.shape[0]
    de = ef.shape[1]
    nblk = e // _BE
    dm = msgs.shape[1]
    wa = pu['W1'][:d_node].astype(jnp.bfloat16)
    wb = pu['W1'][d_node:2 * d_node].astype(jnp.bfloat16)
    wc = pu['W1'][2 * d_node:2 * d_node + pe['W2'].shape[1]]
    wd = pu['W1'][2 * d_node + pe['W2'].shape[1]:].astype(jnp.bfloat16)
    w2u = pu['W2'].astype(jnp.bfloat16)
    p1 = pp['W1'].astype(jnp.bfloat16)
    p2 = pp['W2'].astype(jnp.bfloat16)
    e2c = pe['W2'] @ wc
    bc = (pe['b2'] @ wc + pu['b1']).reshape(1, -1)
    dh = pu['W1'].shape[1]
    do = pu['W2'].shape[1]
    return pl.pallas_call(
        _edge_update_body,
        grid=(nblk,),
        in_specs=[
            pl.BlockSpec((_BE, de), lambda i: (i, 0)),
            pl.BlockSpec((_BE, d_node), lambda i: (i, 0)),
            pl.BlockSpec((_BE, d_node), lambda i: (i + nblk, 0)),
            pl.BlockSpec((_BE, dm), lambda i: (i, 0)),
            _wspec(pe['W1'].shape),
            _wspec((1, pe['W1'].shape[1])),
            _wspec(e2c.shape),
            _wspec(wa.shape),
            _wspec(wb.shape),
            _wspec(wd.shape),
            _wspec((1, dh)),
            _wspec(w2u.shape),
            _wspec((1, do)),
            _wspec(p1.shape),
            _wspec((1, p1.shape[1])),
            _wspec(p2.shape),
            _wspec((1, 1)),
        ],
        out_specs=[
            pl.BlockSpec((_BE, do), lambda i: (i, 0)),
            pl.BlockSpec((_BE, 1), lambda i: (i, 0)),
        ],
        out_shape=[
            jax.ShapeDtypeStruct((e, do), jnp.float32),
            jax.ShapeDtypeStruct((e, 1), jnp.float32),
        ],
    )(ef, gathered, gathered, msgs, pe['W1'], pe['b1'].reshape(1, -1), e2c,
      wa, wb, wd, bc, w2u, pu['b2'].reshape(1, -1),
      p1, pp['b1'].reshape(1, -1), p2, pp['b2'].reshape(1, 1))


_RB = 5
_RBG = 10


def _sc_gather(table, idx):
    b = idx.shape[0]
    d = table.shape[1]
    dt = table.dtype
    b_per_w = b // _NW
    chunks = b_per_w // _K
    groups = chunks // _RBG
    mesh = plsc.VectorSubcoreMesh(core_axis_name="c", subcore_axis_name="s")

    @functools.partial(
        pl.kernel,
        mesh=mesh,
        out_type=jax.ShapeDtypeStruct((b, d), dt),
        scratch_types=(
            [pltpu.VMEM((b_per_w,), jnp.int32)]
            + [pltpu.VMEM((_K, d), dt)] * _RBG
            + [pltpu.SemaphoreType.DMA] * (2 * _RBG)
        ),
    )
    def k(table_hbm, idx_hbm, out_hbm, idx_v, *bufs):
        rows = bufs[:_RBG]
        sg = bufs[_RBG:2 * _RBG]
        sw = bufs[2 * _RBG:]
        wid = lax.axis_index("s") * _NC + lax.axis_index("c")
        base = wid * b_per_w
        pltpu.sync_copy(idx_hbm.at[pl.ds(base, b_per_w)], idx_v)

        def body(g, carry):
            descs = []
            for bb in range(_RBG):
                c = g * _RBG + bb
                off = base + c * _K

                @pl.when(g > 0)
                def _drain_wb(bb=bb, off=off):
                    pltpu.make_async_copy(
                        rows[bb], out_hbm.at[pl.ds(off, _K)], sw[bb]).wait()

                descs.append(pltpu.async_copy(
                    table_hbm.at[idx_v.at[pl.ds(c * _K, _K)]], rows[bb],
                    sg[bb]))
            for bb in range(_RBG):
                c = g * _RBG + bb
                off = base + c * _K
                descs[bb].wait()
                pltpu.async_copy(rows[bb], out_hbm.at[pl.ds(off, _K)],
                                 sw[bb])
            return carry

        lax.fori_loop(0, groups, body, 0)
        for bb in range(_RBG):
            pltpu.make_async_copy(
                rows[bb], out_hbm.at[pl.ds(base, _K)], sw[bb]).wait()

    return k(table, idx)


def _sc_scatter_add(msgs, idx, n):
    e, d = msgs.shape
    per_core = e // _NC
    per_tile = per_core // _NS
    k_sc = 40
    chunks = per_tile // k_sc
    nz = max(t for t in range(1, _NS + 1) if n % t == 0 and (n // t) % 8 == 0)
    stripe_rows = n // nz
    zeros_init = jnp.zeros((stripe_rows, d), jnp.float32)
    mesh = plsc.VectorSubcoreMesh(core_axis_name="c", subcore_axis_name="s")

    groups = chunks // _RB

    @functools.partial(
        pl.kernel,
        mesh=mesh,
        out_type=jax.ShapeDtypeStruct((_NC, n, d), jnp.float32),
        scratch_types=(
            [pltpu.VMEM((k_sc,), jnp.int32)] * _RB
            + [pltpu.VMEM((k_sc, d), jnp.float32)] * _RB
            + [pltpu.VMEM_SHARED((n, d), jnp.float32)]
            + [pltpu.SemaphoreType.DMA] * (3 * _RB)
        ),
    )
    def k(msg_hbm, idx_hbm, zero_hbm, out_hbm, *bufs):
        idxs = bufs[:_RB]
        rows = bufs[_RB:2 * _RB]
        acc = bufs[2 * _RB]
        si = bufs[2 * _RB + 1:2 * _RB + 1 + _RB]
        sm = bufs[2 * _RB + 1 + _RB:2 * _RB + 1 + 2 * _RB]
        sa = bufs[2 * _RB + 1 + 2 * _RB:]
        cid = lax.axis_index("c")
        sid = lax.axis_index("s")
        stripe = pl.ds(sid * stripe_rows, stripe_rows)

        @pl.when(sid < nz)
        def _init():
            pltpu.sync_copy(zero_hbm, acc.at[stripe])

        plsc.subcore_barrier()
        base = cid * per_core + sid * per_tile

        def body(g, carry):
            di = []
            dm = []
            for bb in range(_RB):
                c = g * _RB + bb
                off = base + c * k_sc

                @pl.when(g > 0)
                def _drain_add(bb=bb):
                    pltpu.make_async_copy(
                        rows[bb], acc.at[idxs[bb]], sa[bb]).wait()

                di.append(pltpu.async_copy(
                    idx_hbm.at[pl.ds(off, k_sc)], idxs[bb], si[bb]))
                dm.append(pltpu.async_copy(
                    msg_hbm.at[pl.ds(off, k_sc)], rows[bb], sm[bb]))
            for bb in range(_RB):
                di[bb].wait()
                dm[bb].wait()
                pltpu.async_copy(rows[bb], acc.at[idxs[bb]], sa[bb],
                                 add=True)
            return carry

        lax.fori_loop(0, groups, body, 0)
        for bb in range(_RB):
            pltpu.make_async_copy(rows[bb], acc.at[idxs[bb]], sa[bb]).wait()
        plsc.subcore_barrier()

        @pl.when(sid < nz)
        def _writeback():
            pltpu.sync_copy(acc.at[stripe], out_hbm.at[cid].at[stripe])

    return k(msgs, idx, zeros_init)


def kernel(node_features, edge_features, edge_index, num_nodes, params):
    n, d_node = node_features.shape
    e = edge_features.shape[0]

    idx_all = edge_index.reshape(-1).astype(jnp.int32)
    scat_idx = (edge_index[1] % num_nodes).astype(jnp.int32)

    node_state = _node_encoder(node_features, params['node_encoder'])

    gathered = _sc_gather(node_state, idx_all)
    messages, messages_bf = _messages(edge_features, gathered,
                                      params['edge_encoder'],
                                      params['message_encoder'], d_node)

    agg_parts = _sc_scatter_add(messages, scat_idx, n)

    updated_node_state = _node_update(node_state, agg_parts,
                                      params['node_updater'])

    gathered2 = _sc_gather(updated_node_state, idx_all)
    updated_edge_state, logits = _edge_update(
        edge_features, gathered2, messages_bf, params['edge_encoder'],
        params['edge_updater'], params['predictor'], d_node)

    return (logits.reshape(e), updated_node_state, updated_edge_state,
            messages)
